# baseline (speedup 1.0000x reference)
import functools

import jax
import jax.numpy as jnp
from jax import lax
from jax.experimental import pallas as pl
from jax.experimental.pallas import tpu as pltpu
from jax.experimental.pallas import tpu_sc as plsc

_NUM_CORES = 2
_NUM_SUBCORES = 16
_NUM_WORKERS = _NUM_CORES * _NUM_SUBCORES
_LANES = 16
_CHUNK = 128


@functools.lru_cache(maxsize=None)
def _make_gather(batch: int, half: int):
    b_per_w = batch // _NUM_WORKERS
    n_chunks = b_per_w // _CHUNK
    mesh = plsc.VectorSubcoreMesh(core_axis_name="c", subcore_axis_name="s")

    @functools.partial(
        pl.kernel,
        mesh=mesh,
        out_type=jax.ShapeDtypeStruct((batch, 128), jnp.float32),
        compiler_params=pltpu.CompilerParams(needs_layout_passes=False),
        scratch_types=[
            pltpu.VMEM((b_per_w,), jnp.int32),
            pltpu.VMEM((n_chunks, _CHUNK), jnp.int32),
            pltpu.VMEM((n_chunks, _CHUNK), jnp.int32),
            pltpu.VMEM((_CHUNK, 128), jnp.float32),
            pltpu.VMEM((_CHUNK, 128), jnp.float32),
            pltpu.VMEM((b_per_w, 128), jnp.float32),
            pltpu.SemaphoreType.DMA,
        ],
    )
    def gather_kernel(tabA, tabB, idx_hbm, out_hbm, idx_v, ia_v, ib_v,
                      rowsA, rowsB, out_v, sem):
        wid = lax.axis_index("s") * _NUM_CORES + lax.axis_index("c")
        base = wid * b_per_w
        iota = lax.iota(jnp.int32, _LANES)
        pltpu.sync_copy(idx_hbm.at[pl.ds(base, b_per_w)], idx_v)

        def prep_body(k, _):
            v = idx_v[pl.ds(k * _LANES, _LANES)]
            j = k // (_CHUNK // _LANES)
            o = (k % (_CHUNK // _LANES)) * _LANES
            ia_v[j, pl.ds(o, _LANES)] = jnp.minimum(v, half - 1)
            ib_v[j, pl.ds(o, _LANES)] = jnp.maximum(v - half, 0)
            return ()

        lax.fori_loop(0, b_per_w // _LANES, prep_body, (), unroll=False)

        for j in range(n_chunks):
            ca = pltpu.async_copy(tabA.at[ia_v.at[j]], rowsA, sem)
            cb = pltpu.async_copy(tabB.at[ib_v.at[j]], rowsB, sem)
            ca.wait()
            cb.wait()

            def sel_body(k, _):
                row = jnp.zeros((_LANES,), jnp.int32) + k
                vk = plsc.load_gather(idx_v, [row + j * _CHUNK])
                m = vk < half
                for c0 in range(128 // _LANES):
                    cols = c0 * _LANES + iota
                    va = plsc.load_gather(rowsA, [row, cols])
                    vb = plsc.load_gather(rowsB, [row, cols])
                    out_v[j * _CHUNK + k, pl.ds(c0 * _LANES, _LANES)] = (
                        jnp.where(m, va, vb)
                    )
                return ()

            lax.fori_loop(0, _CHUNK, sel_body, (), unroll=False)

        pltpu.sync_copy(out_v, out_hbm.at[pl.ds(base, b_per_w)])

    return gather_kernel


def kernel(input_x, Emb):
    batch = input_x.shape[1]
    n, dim = Emb.shape
    half = n // 2
    tabA = jnp.pad(Emb[:half], ((0, 0), (0, 128 - dim)))
    tabB = jnp.pad(Emb[half:], ((0, 0), (0, 128 - dim)))
    idx = input_x.reshape(batch)
    out = _make_gather(batch, half)(tabA, tabB, idx)
    return out[:, :dim]
